# Initial kernel scaffold; baseline (speedup 1.0000x reference)
import jax
import jax.numpy as jnp
from jax.experimental import pallas as pl


def kernel(xs, ys, x_lengths, y_lengths, P, do_nothing_ij):
    raise NotImplementedError("write your pallas kernel here")


# speedup vs baseline: 81.6928x; 81.6928x over previous
import functools

import jax
import jax.numpy as jnp
from jax import lax
from jax.experimental import pallas as pl
from jax.experimental.pallas import tpu as pltpu

B = 8
L = 512


def _dp_kernel(xs_ref, ys_ref, xlen_ref, ylen_ref, p_ref, dn_ref,
               out_ref, m2_ref, cx_ref):
    f32 = jnp.float32
    P = p_ref[...]
    mx = jnp.max(P, axis=1, keepdims=True)
    e = jnp.exp(P - mx)
    Q = 1.0 - e / jnp.sum(e, axis=1, keepdims=True)

    dn0 = dn_ref[0]
    dn1 = dn_ref[1]

    iota_s = lax.broadcasted_iota(jnp.int32, (L, L), 0)
    iota_l = lax.broadcasted_iota(jnp.int32, (L, L), 1)

    qdn = jnp.sum(jnp.where(iota_s == dn0, Q, 0.0), axis=0, keepdims=True)
    dn1_oh = (lax.broadcasted_iota(jnp.int32, (1, L), 1) == dn1).astype(f32)

    cy_rows = []
    for b in range(B):
        xb = xs_ref[pl.ds(b, 1), :]
        ohxT = (jnp.broadcast_to(xb, (L, L)) == iota_s).astype(f32)
        Qx = lax.dot_general(ohxT, Q, (((0,), (0,)), ((), ())),
                             preferred_element_type=f32)
        yb = ys_ref[pl.ds(b, 1), :]
        ohyT = (jnp.broadcast_to(yb, (L, L)) == iota_s).astype(f32)
        Mb = lax.dot_general(Qx, ohyT, (((1,), (0,)), ((), ())),
                             preferred_element_type=f32)
        cyb = lax.dot_general(qdn, ohyT, (((1,), (0,)), ((), ())),
                              preferred_element_type=f32)
        cxb = jnp.sum(jnp.where(iota_l == dn1, Qx, 0.0), axis=1,
                      keepdims=True)
        cx_ref[:, pl.ds(b, 1), :] = cxb.reshape(L, 1, 1)
        m2_ref[:, pl.ds(b, 1), :] = (Mb - cyb).reshape(L, 1, L)
        cy_rows.append(cyb)

    cy_all = jnp.concatenate(cy_rows, axis=0)

    lane = lax.broadcasted_iota(jnp.int32, (B, L), 1)
    xe = xlen_ref[...] - 1
    ye = ylen_ref[...] - 1
    sel_col = lane == jnp.broadcast_to(ye - 1, (B, L))
    ccy_mask = lane <= jnp.broadcast_to(ye - 1, (B, L))
    ccy_at = jnp.sum(jnp.where(ccy_mask, cy_all, 0.0), axis=1, keepdims=True)

    def row(i, carry):
        ev, e0, acc = carry
        cx = cx_ref[pl.ds(i - 1, 1), :, :].reshape(B, 1)
        m2 = m2_ref[pl.ds(i - 1, 1), :, :].reshape(B, L)
        esh = pltpu.roll(ev, shift=1, axis=1)
        esh = jnp.where(lane == 0, jnp.broadcast_to(e0, (B, L)), esh)
        t = jnp.minimum(esh + m2, ev + cx)
        e0n = e0 + cx
        for k in (1, 2, 4, 8, 16, 32, 64, 128, 256):
            r = pltpu.roll(t, shift=k, axis=1)
            t = jnp.minimum(t, jnp.where(lane >= k, r, jnp.inf))
        evn = jnp.minimum(t, e0n)
        hit = sel_col & jnp.broadcast_to(xe == i, (B, L))
        acc = acc + jnp.sum(jnp.where(hit, evn, 0.0), axis=1, keepdims=True)
        return evn, e0n, acc

    init = (jnp.zeros((B, L), f32), jnp.zeros((B, 1), f32), jnp.zeros((B, 1), f32))
    _, _, acc = lax.fori_loop(1, L + 1, row, init)

    total = jnp.sum(acc + ccy_at, axis=0, keepdims=True)
    out_ref[...] = total * (1.0 / B)


@jax.jit
def kernel(xs, ys, x_lengths, y_lengths, P, do_nothing_ij):
    out = pl.pallas_call(
        _dp_kernel,
        out_shape=jax.ShapeDtypeStruct((1, 1), jnp.float32),
        in_specs=[
            pl.BlockSpec((B, L), lambda: (0, 0)),
            pl.BlockSpec((B, L), lambda: (0, 0)),
            pl.BlockSpec((B, 1), lambda: (0, 0)),
            pl.BlockSpec((B, 1), lambda: (0, 0)),
            pl.BlockSpec((L, L), lambda: (0, 0)),
            pl.BlockSpec(memory_space=pltpu.SMEM),
        ],
        out_specs=pl.BlockSpec((1, 1), lambda: (0, 0)),
        scratch_shapes=[
            pltpu.VMEM((L, B, L), jnp.float32),
            pltpu.VMEM((L, B, 1), jnp.float32),
        ],
    )(xs, ys, x_lengths.reshape(B, 1), y_lengths.reshape(B, 1), P,
      do_nothing_ij)
    return out[0, 0]


# speedup vs baseline: 96.0858x; 1.1762x over previous
import functools

import jax
import jax.numpy as jnp
from jax import lax
from jax.experimental import pallas as pl
from jax.experimental.pallas import tpu as pltpu

B = 8
L = 512


def _dp_kernel(xs_ref, ys_ref, xlen_ref, ylen_ref, p_ref, dn_ref,
               out_ref, m2_ref, cx_ref):
    f32 = jnp.float32
    P = p_ref[...]
    mx = jnp.max(P, axis=1, keepdims=True)
    e = jnp.exp(P - mx)
    Q = 1.0 - e / jnp.sum(e, axis=1, keepdims=True)

    dn0 = dn_ref[0]
    dn1 = dn_ref[1]

    iota_s = lax.broadcasted_iota(jnp.int32, (L, L), 0)
    iota_l = lax.broadcasted_iota(jnp.int32, (L, L), 1)

    qdn = jnp.sum(jnp.where(iota_s == dn0, Q, 0.0), axis=0, keepdims=True)
    dn1_oh = (lax.broadcasted_iota(jnp.int32, (1, L), 1) == dn1).astype(f32)

    cy_rows = []
    for b in range(B):
        xb = xs_ref[pl.ds(b, 1), :]
        ohxT = (jnp.broadcast_to(xb, (L, L)) == iota_s).astype(f32)
        Qx = lax.dot_general(ohxT, Q, (((0,), (0,)), ((), ())),
                             preferred_element_type=f32)
        yb = ys_ref[pl.ds(b, 1), :]
        ohyT = (jnp.broadcast_to(yb, (L, L)) == iota_s).astype(f32)
        Mb = lax.dot_general(Qx, ohyT, (((1,), (0,)), ((), ())),
                             preferred_element_type=f32)
        cyb = lax.dot_general(qdn, ohyT, (((1,), (0,)), ((), ())),
                              preferred_element_type=f32)
        cxb = jnp.sum(jnp.where(iota_l == dn1, Qx, 0.0), axis=1,
                      keepdims=True)
        cx_ref[:, pl.ds(b, 1), :] = cxb.reshape(L, 1, 1)
        m2_ref[:, pl.ds(b, 1), :] = (Mb - cyb).reshape(L, 1, L)
        cy_rows.append(cyb)

    cy_all = jnp.concatenate(cy_rows, axis=0)

    lane = lax.broadcasted_iota(jnp.int32, (B, L), 1)
    xe = xlen_ref[...] - 1
    ye = ylen_ref[...] - 1
    sel_col = lane == jnp.broadcast_to(ye - 1, (B, L))
    ccy_mask = lane <= jnp.broadcast_to(ye - 1, (B, L))
    ccy_at = jnp.sum(jnp.where(ccy_mask, cy_all, 0.0), axis=1, keepdims=True)

    def one_row(i, ev, e0, acc):
        cx = cx_ref[pl.ds(i - 1, 1), :, :].reshape(B, 1)
        m2 = m2_ref[pl.ds(i - 1, 1), :, :].reshape(B, L)
        esh = pltpu.roll(ev, shift=1, axis=1)
        esh = jnp.where(lane == 0, jnp.broadcast_to(e0, (B, L)), esh)
        t = jnp.minimum(esh + m2, ev + cx)
        e0n = e0 + cx
        for k in (1, 2, 4, 8, 16, 32, 64, 128, 256):
            r = pltpu.roll(t, shift=k, axis=1)
            t = jnp.minimum(t, jnp.where(lane >= k, r, jnp.inf))
        evn = jnp.minimum(t, e0n)
        hit = sel_col & jnp.broadcast_to(xe == i, (B, L))
        acc = acc + jnp.sum(jnp.where(hit, evn, 0.0), axis=1, keepdims=True)
        return evn, e0n, acc

    U = 4

    def step(s, carry):
        ev, e0, acc = carry
        for r in range(1, U + 1):
            ev, e0, acc = one_row(U * s + r, ev, e0, acc)
        return ev, e0, acc

    n_steps = (jnp.max(xe) + (U - 1)) // U
    init = (jnp.zeros((B, L), f32), jnp.zeros((B, 1), f32), jnp.zeros((B, 1), f32))
    _, _, acc = lax.fori_loop(0, n_steps, step, init)

    total = jnp.sum(acc + ccy_at, axis=0, keepdims=True)
    out_ref[...] = total * (1.0 / B)


@jax.jit
def kernel(xs, ys, x_lengths, y_lengths, P, do_nothing_ij):
    out = pl.pallas_call(
        _dp_kernel,
        out_shape=jax.ShapeDtypeStruct((1, 1), jnp.float32),
        in_specs=[
            pl.BlockSpec((B, L), lambda: (0, 0)),
            pl.BlockSpec((B, L), lambda: (0, 0)),
            pl.BlockSpec((B, 1), lambda: (0, 0)),
            pl.BlockSpec((B, 1), lambda: (0, 0)),
            pl.BlockSpec((L, L), lambda: (0, 0)),
            pl.BlockSpec(memory_space=pltpu.SMEM),
        ],
        out_specs=pl.BlockSpec((1, 1), lambda: (0, 0)),
        scratch_shapes=[
            pltpu.VMEM((L, B, L), jnp.float32),
            pltpu.VMEM((L, B, 1), jnp.float32),
        ],
    )(xs, ys, x_lengths.reshape(B, 1), y_lengths.reshape(B, 1), P,
      do_nothing_ij)
    return out[0, 0]


# speedup vs baseline: 441.9995x; 4.6001x over previous
import jax
import jax.numpy as jnp
from jax import lax
from jax.experimental import pallas as pl
from jax.experimental.pallas import tpu as pltpu

B = 8
L = 512


def _dp_kernel(xs_ref, ys_ref, xlen_ref, ylen_ref, p_ref, dn_ref,
               out_ref, t_ref):
    f32 = jnp.float32
    P = p_ref[...]
    mx = jnp.max(P, axis=1, keepdims=True)
    e = jnp.exp(P - mx)
    Q = 1.0 - e / jnp.sum(e, axis=1, keepdims=True)

    dn0 = dn_ref[0]
    dn1 = dn_ref[1]

    iota_s = lax.broadcasted_iota(jnp.int32, (L, L), 0)
    qdn = jnp.sum(jnp.where(iota_s == dn0, Q, 0.0), axis=0, keepdims=True)
    dn1_oh = (lax.broadcasted_iota(jnp.int32, (1, L), 1) == dn1).astype(f32)

    cy_rows = []
    cx_rows = []
    for b in range(B):
        xb = xs_ref[pl.ds(b, 1), :]
        ohxT = (jnp.broadcast_to(xb, (L, L)) == iota_s).astype(f32)
        Qx = lax.dot_general(ohxT, Q, (((0,), (0,)), ((), ())),
                             preferred_element_type=f32)
        yb = ys_ref[pl.ds(b, 1), :]
        ohyT = (jnp.broadcast_to(yb, (L, L)) == iota_s).astype(f32)
        mb = lax.dot_general(Qx, ohyT, (((1,), (0,)), ((), ())),
                             preferred_element_type=f32)
        cyb = lax.dot_general(qdn, ohyT, (((1,), (0,)), ((), ())),
                              preferred_element_type=f32)
        cy_rows.append(cyb)
        cx_rows.append(lax.dot_general(dn1_oh, Qx, (((1,), (1,)), ((), ())),
                                       preferred_element_type=f32))
        skewed = pltpu.roll(mb - cyb, 0, 1, stride=1, stride_axis=0)
        tb = jnp.transpose(skewed)
        t_ref[:, pl.ds(b, 1), :] = tb.reshape(L, 1, L)

    cy_all = jnp.concatenate(cy_rows, axis=0)
    cxl = jnp.concatenate(cx_rows, axis=0)

    lane = lax.broadcasted_iota(jnp.int32, (B, L), 1)
    xe = xlen_ref[...] - 1
    ye = ylen_ref[...] - 1
    de = xe + ye
    lanekx = lane == jnp.broadcast_to(xe - 1, (B, L))
    ccy_mask = lane <= jnp.broadcast_to(ye - 1, (B, L))
    ccy_at = jnp.sum(jnp.where(ccy_mask, cy_all, 0.0), axis=1, keepdims=True)

    inf = jnp.float32(jnp.inf)

    def one_diag(d, f_prev, r_prev, acc):
        c = (d - 2) & (L - 1)
        tslab = t_ref[pl.ds(c, 1), :, :].reshape(B, L)
        r1 = pltpu.roll(f_prev, 1, 1)
        b0 = jnp.where(d <= L + 1, jnp.float32(0.0), inf)
        r1 = jnp.where(lane == 0, b0, r1)
        f_new = jnp.minimum(jnp.minimum(r_prev + tslab, f_prev), r1 + cxl)
        hit = lanekx & jnp.broadcast_to(de == d, (B, L))
        acc = jnp.where(hit, f_new, acc)
        return f_new, r1, acc

    U = 4

    def step(s, carry):
        f_prev, r_prev, acc = carry
        for r in range(1, U + 1):
            f_prev, r_prev, acc = one_diag(U * s + r, f_prev, r_prev, acc)
        return f_prev, r_prev, acc

    n_steps = (jnp.max(de) + (U - 1)) // U
    init = (jnp.full((B, L), inf, f32), jnp.full((B, L), inf, f32),
            jnp.zeros((B, L), f32))
    _, _, acc = lax.fori_loop(0, n_steps, step, init)

    phi = jnp.sum(acc, axis=1, keepdims=True)
    total = jnp.sum(phi + ccy_at, axis=0, keepdims=True)
    out_ref[...] = total * (1.0 / B)


@jax.jit
def kernel(xs, ys, x_lengths, y_lengths, P, do_nothing_ij):
    out = pl.pallas_call(
        _dp_kernel,
        out_shape=jax.ShapeDtypeStruct((1, 1), jnp.float32),
        in_specs=[
            pl.BlockSpec((B, L), lambda: (0, 0)),
            pl.BlockSpec((B, L), lambda: (0, 0)),
            pl.BlockSpec((B, 1), lambda: (0, 0)),
            pl.BlockSpec((B, 1), lambda: (0, 0)),
            pl.BlockSpec((L, L), lambda: (0, 0)),
            pl.BlockSpec(memory_space=pltpu.SMEM),
        ],
        out_specs=pl.BlockSpec((1, 1), lambda: (0, 0)),
        scratch_shapes=[
            pltpu.VMEM((L, B, L), jnp.float32),
        ],
    )(xs, ys, x_lengths.reshape(B, 1), y_lengths.reshape(B, 1), P,
      do_nothing_ij)
    return out[0, 0]
